# baseline (speedup 1.0000x reference)
import jax
import jax.numpy as jnp
from jax.experimental import pallas as pl
from jax.experimental.pallas import tpu as pltpu

_MARGIN = 1.0
_N = 2048
_BLK = 256
_MSE_BLK = 128


def _body(emb_i_ref, emb_all_ref, tgt_ref, tgt_i_ref,
          o1_ref, t1_ref, o2_ref, t2_ref,
          pair_ref, sse1_ref, sse2_ref, sqs_ref):
    pid = pl.program_id(0)

    e_i = emb_i_ref[...]
    e_all = emb_all_ref[...]

    dot = jax.lax.dot_general(
        e_i, e_all,
        dimension_numbers=(((1,), (1,)), ((), ())),
        preferred_element_type=jnp.float32,
        precision=jax.lax.Precision.DEFAULT,
    )

    @pl.when(pid == 0)
    def _sq():
        sq_all = jnp.sum(e_all * e_all, axis=1, keepdims=True)
        sqs_ref[...] = jnp.transpose(sq_all)

    sq_all_row = sqs_ref[...]
    sq_i = jnp.transpose(sqs_ref[:, pl.ds(pid * _BLK, _BLK)])

    d2 = jnp.maximum(sq_i + sq_all_row - 2.0 * dot, 0.0)

    tgt_all = tgt_ref[...]
    tgt_i = tgt_i_ref[...]
    same = jnp.transpose(tgt_i) == tgt_all

    neg_vals = jnp.maximum(_MARGIN - jnp.sqrt(d2), 0.0)
    pair_part = jnp.sum(jnp.where(same, d2, neg_vals * neg_vals))

    r1 = o1_ref[...] - t1_ref[...]
    r2 = o2_ref[...] - t2_ref[...]
    sse1_part = jnp.sum(r1 * r1)
    sse2_part = jnp.sum(r2 * r2)

    @pl.when(pid == 0)
    def _init():
        pair_ref[...] = pair_part[None, None]
        sse1_ref[...] = sse1_part[None, None]
        sse2_ref[...] = sse2_part[None, None]

    @pl.when(pid != 0)
    def _acc():
        pair_ref[...] += pair_part[None, None]
        sse1_ref[...] += sse1_part[None, None]
        sse2_ref[...] += sse2_part[None, None]


def kernel(feature1, feature2, output1, output2, target1, target2, label):
    B, D = output1.shape
    emb = jnp.concatenate([feature1, feature2], axis=0)
    tgt = jnp.concatenate([label[0], label[1]], axis=0)[None, :]

    n_steps = _N // _BLK
    scalar = jax.ShapeDtypeStruct((1, 1), jnp.float32)

    pair2, sse1, sse2 = pl.pallas_call(
        _body,
        grid=(n_steps,),
        in_specs=[
            pl.BlockSpec((_BLK, D), lambda i: (i, 0)),
            pl.BlockSpec((_N, D), lambda i: (0, 0)),
            pl.BlockSpec((1, _N), lambda i: (0, 0)),
            pl.BlockSpec((1, _BLK), lambda i: (0, i)),
            pl.BlockSpec((_MSE_BLK, D), lambda i: (i, 0)),
            pl.BlockSpec((_MSE_BLK, D), lambda i: (i, 0)),
            pl.BlockSpec((_MSE_BLK, D), lambda i: (i, 0)),
            pl.BlockSpec((_MSE_BLK, D), lambda i: (i, 0)),
        ],
        out_specs=[
            pl.BlockSpec((1, 1), lambda i: (0, 0)),
            pl.BlockSpec((1, 1), lambda i: (0, 0)),
            pl.BlockSpec((1, 1), lambda i: (0, 0)),
        ],
        out_shape=[scalar, scalar, scalar],
        scratch_shapes=[pltpu.VMEM((1, _N), jnp.float32)],
    )(emb, emb, tgt, tgt, output1, target1, output2, target2)

    n_pairs = jnp.float32(_N * (_N - 1) / 2)
    denom = jnp.float32(B * D)
    loss1 = sse1[0, 0] / denom
    loss2 = sse2[0, 0] / denom
    loss_mean = 0.5 * pair2[0, 0] / n_pairs
    losses = loss_mean + (loss1 + loss2) / 2.0
    return (losses, loss1, loss2, loss_mean)


# speedup vs baseline: 1.0702x; 1.0702x over previous
import numpy as np
import jax
import jax.numpy as jnp
from jax.experimental import pallas as pl
from jax.experimental.pallas import tpu as pltpu

_MARGIN = 1.0
_N = 2048
_BLK = 512
_NB = _N // _BLK
_MSE_BLK = 256

_IMAP = np.array([i for i in range(_NB) for j in range(i, _NB)], dtype=np.int32)
_JMAP = np.array([j for i in range(_NB) for j in range(i, _NB)], dtype=np.int32)
_NTILES = len(_IMAP)


def _body(imap_ref, jmap_ref,
          ei_ref, ej_ref, tgt_i_ref, tgt_j_ref,
          o1_ref, t1_ref, o2_ref, t2_ref,
          pair_ref, sse1_ref, sse2_ref, sqs_ref):
    p = pl.program_id(0)
    i = imap_ref[p]
    j = jmap_ref[p]

    e_i = ei_ref[...]
    e_j = ej_ref[...]

    @pl.when(i == 0)
    def _fill():
        sq_j = jnp.sum(e_j * e_j, axis=1, keepdims=True)
        sqs_ref[:, pl.ds(j * _BLK, _BLK)] = jnp.transpose(sq_j)

    sq_j_row = sqs_ref[:, pl.ds(j * _BLK, _BLK)]
    sq_i_col = jnp.transpose(sqs_ref[:, pl.ds(i * _BLK, _BLK)])

    dot = jax.lax.dot_general(
        e_i, e_j,
        dimension_numbers=(((1,), (1,)), ((), ())),
        preferred_element_type=jnp.float32,
        precision=jax.lax.Precision.DEFAULT,
    )

    d2 = jnp.maximum(sq_i_col + sq_j_row - 2.0 * dot, 0.0)

    same = jnp.transpose(tgt_i_ref[...]) == tgt_j_ref[...]
    neg_vals = jnp.maximum(_MARGIN - jnp.sqrt(d2), 0.0)
    tsum = jnp.sum(jnp.where(same, d2, neg_vals * neg_vals))
    pair_part = jnp.where(i == j, 0.5, 1.0) * tsum

    @pl.when(p == 0)
    def _init():
        pair_ref[...] = jnp.zeros_like(pair_ref)
        sse1_ref[...] = jnp.zeros_like(sse1_ref)
        sse2_ref[...] = jnp.zeros_like(sse2_ref)

    pair_ref[...] += pair_part[None, None]

    @pl.when(i == j)
    def _mse():
        r1 = o1_ref[...] - t1_ref[...]
        r2 = o2_ref[...] - t2_ref[...]
        sse1_ref[...] += jnp.sum(r1 * r1)[None, None]
        sse2_ref[...] += jnp.sum(r2 * r2)[None, None]


def kernel(feature1, feature2, output1, output2, target1, target2, label):
    B, D = output1.shape
    emb = jnp.concatenate([feature1, feature2], axis=0)
    tgt = jnp.concatenate([label[0], label[1]], axis=0)[None, :]

    scalar = jax.ShapeDtypeStruct((1, 1), jnp.float32)

    grid_spec = pltpu.PrefetchScalarGridSpec(
        num_scalar_prefetch=2,
        grid=(_NTILES,),
        in_specs=[
            pl.BlockSpec((_BLK, D), lambda p, im, jm: (im[p], 0)),
            pl.BlockSpec((_BLK, D), lambda p, im, jm: (jm[p], 0)),
            pl.BlockSpec((1, _BLK), lambda p, im, jm: (0, im[p])),
            pl.BlockSpec((1, _BLK), lambda p, im, jm: (0, jm[p])),
            pl.BlockSpec((_MSE_BLK, D), lambda p, im, jm: (im[p], 0)),
            pl.BlockSpec((_MSE_BLK, D), lambda p, im, jm: (im[p], 0)),
            pl.BlockSpec((_MSE_BLK, D), lambda p, im, jm: (im[p], 0)),
            pl.BlockSpec((_MSE_BLK, D), lambda p, im, jm: (im[p], 0)),
        ],
        out_specs=[
            pl.BlockSpec((1, 1), lambda p, im, jm: (0, 0)),
            pl.BlockSpec((1, 1), lambda p, im, jm: (0, 0)),
            pl.BlockSpec((1, 1), lambda p, im, jm: (0, 0)),
        ],
        scratch_shapes=[pltpu.VMEM((1, _N), jnp.float32)],
    )

    pair, sse1, sse2 = pl.pallas_call(
        _body,
        grid_spec=grid_spec,
        out_shape=[scalar, scalar, scalar],
    )(jnp.asarray(_IMAP), jnp.asarray(_JMAP),
      emb, emb, tgt, tgt, output1, target1, output2, target2)

    n_pairs = jnp.float32(_N * (_N - 1) / 2)
    denom = jnp.float32(B * D)
    loss1 = sse1[0, 0] / denom
    loss2 = sse2[0, 0] / denom
    loss_mean = pair[0, 0] / n_pairs
    losses = loss_mean + (loss1 + loss2) / 2.0
    return (losses, loss1, loss2, loss_mean)


# speedup vs baseline: 1.1747x; 1.0977x over previous
import numpy as np
import jax
import jax.numpy as jnp
from jax.experimental import pallas as pl
from jax.experimental.pallas import tpu as pltpu

_MARGIN = 1.0
_N = 2048
_BLK = 1024
_NB = _N // _BLK
_MSE_BLK = 512

_IMAP = np.array([i for i in range(_NB) for j in range(i, _NB)], dtype=np.int32)
_JMAP = np.array([j for i in range(_NB) for j in range(i, _NB)], dtype=np.int32)
_NTILES = len(_IMAP)


def _body(imap_ref, jmap_ref,
          ei_ref, ej_ref, tgt_i_ref, tgt_j_ref,
          o1_ref, t1_ref, o2_ref, t2_ref,
          pair_ref, sse1_ref, sse2_ref, sqs_ref):
    p = pl.program_id(0)
    i = imap_ref[p]
    j = jmap_ref[p]

    e_i = ei_ref[...]
    e_j = ej_ref[...]

    @pl.when(i == 0)
    def _fill():
        sq_j = jnp.sum(e_j * e_j, axis=1, keepdims=True)
        sqs_ref[:, pl.ds(j * _BLK, _BLK)] = jnp.transpose(sq_j)

    sq_j_row = sqs_ref[:, pl.ds(j * _BLK, _BLK)]
    sq_i_col = jnp.transpose(sqs_ref[:, pl.ds(i * _BLK, _BLK)])

    dot = jax.lax.dot_general(
        e_i, e_j,
        dimension_numbers=(((1,), (1,)), ((), ())),
        preferred_element_type=jnp.float32,
        precision=jax.lax.Precision.DEFAULT,
    )

    d2 = jnp.maximum(sq_i_col + sq_j_row - 2.0 * dot, 0.0)

    same = jnp.transpose(tgt_i_ref[...]) == tgt_j_ref[...]
    neg_vals = jnp.maximum(_MARGIN - jnp.sqrt(d2), 0.0)
    tsum = jnp.sum(jnp.where(same, d2, neg_vals * neg_vals))
    pair_part = jnp.where(i == j, 0.5, 1.0) * tsum

    @pl.when(p == 0)
    def _init():
        pair_ref[...] = jnp.zeros_like(pair_ref)
        sse1_ref[...] = jnp.zeros_like(sse1_ref)
        sse2_ref[...] = jnp.zeros_like(sse2_ref)

    pair_ref[...] += pair_part[None, None]

    @pl.when(i == j)
    def _mse():
        r1 = o1_ref[...] - t1_ref[...]
        r2 = o2_ref[...] - t2_ref[...]
        sse1_ref[...] += jnp.sum(r1 * r1)[None, None]
        sse2_ref[...] += jnp.sum(r2 * r2)[None, None]


def kernel(feature1, feature2, output1, output2, target1, target2, label):
    B, D = output1.shape
    emb = jnp.concatenate([feature1, feature2], axis=0)
    tgt = jnp.concatenate([label[0], label[1]], axis=0)[None, :]

    scalar = jax.ShapeDtypeStruct((1, 1), jnp.float32)

    grid_spec = pltpu.PrefetchScalarGridSpec(
        num_scalar_prefetch=2,
        grid=(_NTILES,),
        in_specs=[
            pl.BlockSpec((_BLK, D), lambda p, im, jm: (im[p], 0)),
            pl.BlockSpec((_BLK, D), lambda p, im, jm: (jm[p], 0)),
            pl.BlockSpec((1, _BLK), lambda p, im, jm: (0, im[p])),
            pl.BlockSpec((1, _BLK), lambda p, im, jm: (0, jm[p])),
            pl.BlockSpec((_MSE_BLK, D), lambda p, im, jm: (im[p], 0)),
            pl.BlockSpec((_MSE_BLK, D), lambda p, im, jm: (im[p], 0)),
            pl.BlockSpec((_MSE_BLK, D), lambda p, im, jm: (im[p], 0)),
            pl.BlockSpec((_MSE_BLK, D), lambda p, im, jm: (im[p], 0)),
        ],
        out_specs=[
            pl.BlockSpec((1, 1), lambda p, im, jm: (0, 0)),
            pl.BlockSpec((1, 1), lambda p, im, jm: (0, 0)),
            pl.BlockSpec((1, 1), lambda p, im, jm: (0, 0)),
        ],
        scratch_shapes=[pltpu.VMEM((1, _N), jnp.float32)],
    )

    pair, sse1, sse2 = pl.pallas_call(
        _body,
        grid_spec=grid_spec,
        out_shape=[scalar, scalar, scalar],
    )(jnp.asarray(_IMAP), jnp.asarray(_JMAP),
      emb, emb, tgt, tgt, output1, target1, output2, target2)

    n_pairs = jnp.float32(_N * (_N - 1) / 2)
    denom = jnp.float32(B * D)
    loss1 = sse1[0, 0] / denom
    loss2 = sse2[0, 0] / denom
    loss_mean = pair[0, 0] / n_pairs
    losses = loss_mean + (loss1 + loss2) / 2.0
    return (losses, loss1, loss2, loss_mean)


# speedup vs baseline: 1.5517x; 1.3209x over previous
import jax
import jax.numpy as jnp
from jax.experimental import pallas as pl
from jax.experimental.pallas import tpu as pltpu

_MARGIN = 1.0
_H = 1024


def _rowsq(e):
    return jnp.sum(e * e, axis=1, keepdims=True)


def _tile_sum(e_a, e_b, sq_a_col, sq_b_row, lbl_a, lbl_b):
    dot = jax.lax.dot_general(
        e_a, e_b,
        dimension_numbers=(((1,), (1,)), ((), ())),
        preferred_element_type=jnp.float32,
        precision=jax.lax.Precision.DEFAULT,
    )
    d2 = jnp.maximum(sq_a_col + sq_b_row - 2.0 * dot, 0.0)
    same = jnp.transpose(lbl_a) == lbl_b
    neg_vals = jnp.maximum(_MARGIN - jnp.sqrt(d2), 0.0)
    return jnp.sum(jnp.where(same, d2, neg_vals * neg_vals))


def _body(f1_ref, f2_ref, lbl_ref, o1_ref, t1_ref, o2_ref, t2_ref,
          pair_ref, sse1_ref, sse2_ref, sq1_ref, sq2_ref):
    p = pl.program_id(0)

    @pl.when(p == 0)
    def _t00():
        e1 = f1_ref[...]
        sq1_ref[...] = jnp.transpose(_rowsq(e1))
        sq1 = sq1_ref[...]
        l0 = lbl_ref[0:1, :]
        tsum = _tile_sum(e1, e1, jnp.transpose(sq1), sq1, l0, l0)
        pair_ref[...] = (0.5 * tsum)[None, None]
        r1 = o1_ref[...] - t1_ref[...]
        sse1_ref[...] = jnp.sum(r1 * r1)[None, None]

    @pl.when(p == 1)
    def _t01():
        e1 = f1_ref[...]
        e2 = f2_ref[...]
        sq2_ref[...] = jnp.transpose(_rowsq(e2))
        sq1 = sq1_ref[...]
        sq2 = sq2_ref[...]
        l0 = lbl_ref[0:1, :]
        l1 = lbl_ref[1:2, :]
        tsum = _tile_sum(e1, e2, jnp.transpose(sq1), sq2, l0, l1)
        pair_ref[...] += tsum[None, None]
        r2 = o2_ref[...] - t2_ref[...]
        sse2_ref[...] = jnp.sum(r2 * r2)[None, None]

    @pl.when(p == 2)
    def _t11():
        e2 = f2_ref[...]
        sq2 = sq2_ref[...]
        l1 = lbl_ref[1:2, :]
        tsum = _tile_sum(e2, e2, jnp.transpose(sq2), sq2, l1, l1)
        pair_ref[...] += (0.5 * tsum)[None, None]


def kernel(feature1, feature2, output1, output2, target1, target2, label):
    B, D = output1.shape
    scalar = jax.ShapeDtypeStruct((1, 1), jnp.float32)

    full = lambda shape: pl.BlockSpec(shape, lambda p: (0, 0))
    pair, sse1, sse2 = pl.pallas_call(
        _body,
        grid=(3,),
        in_specs=[
            full((_H, D)),
            full((_H, D)),
            full((2, _H)),
            full((_H, D)),
            full((_H, D)),
            full((_H, D)),
            full((_H, D)),
        ],
        out_specs=[full((1, 1)), full((1, 1)), full((1, 1))],
        out_shape=[scalar, scalar, scalar],
        scratch_shapes=[pltpu.VMEM((1, _H), jnp.float32),
                        pltpu.VMEM((1, _H), jnp.float32)],
    )(feature1, feature2, label, output1, target1, output2, target2)

    n = 2 * _H
    n_pairs = jnp.float32(n * (n - 1) / 2)
    denom = jnp.float32(B * D)
    loss1 = sse1[0, 0] / denom
    loss2 = sse2[0, 0] / denom
    loss_mean = pair[0, 0] / n_pairs
    losses = loss_mean + (loss1 + loss2) / 2.0
    return (losses, loss1, loss2, loss_mean)


# speedup vs baseline: 1.6170x; 1.0421x over previous
import jax
import jax.numpy as jnp
from jax.experimental import pallas as pl
from jax.experimental.pallas import tpu as pltpu

_MARGIN = 1.0
_H = 1024


def _rowsq(e):
    return jnp.sum(e * e, axis=1, keepdims=True)


def _tile_sum(e_a, e_b, sq_a_col, sq_b_row, lbl_a, lbl_b):
    dot = jax.lax.dot_general(
        e_a, e_b,
        dimension_numbers=(((1,), (1,)), ((), ())),
        preferred_element_type=jnp.float32,
        precision=jax.lax.Precision.DEFAULT,
    )
    d2 = jnp.maximum(sq_a_col + sq_b_row - 2.0 * dot, 0.0)
    same = jnp.transpose(lbl_a) == lbl_b
    neg_vals = jnp.maximum(_MARGIN - jnp.sqrt(d2), 0.0)
    return jnp.sum(jnp.where(same, d2, neg_vals * neg_vals))


def _body(f1_ref, f2_ref, lbl_ref, o1_ref, t1_ref, o2_ref, t2_ref,
          pair_ref, sse1_ref, sse2_ref, sq1_ref, sq2_ref):
    e1 = f1_ref[...]
    e2 = f2_ref[...]
    sq1_ref[...] = jnp.transpose(_rowsq(e1))
    sq2_ref[...] = jnp.transpose(_rowsq(e2))
    sq1 = sq1_ref[...]
    sq2 = sq2_ref[...]
    l0 = lbl_ref[0:1, :]
    l1 = lbl_ref[1:2, :]
    t00 = _tile_sum(e1, e1, jnp.transpose(sq1), sq1, l0, l0)
    t01 = _tile_sum(e1, e2, jnp.transpose(sq1), sq2, l0, l1)
    t11 = _tile_sum(e2, e2, jnp.transpose(sq2), sq2, l1, l1)
    pair_ref[...] = (0.5 * t00 + t01 + 0.5 * t11)[None, None]
    r1 = o1_ref[...] - t1_ref[...]
    sse1_ref[...] = jnp.sum(r1 * r1)[None, None]
    r2 = o2_ref[...] - t2_ref[...]
    sse2_ref[...] = jnp.sum(r2 * r2)[None, None]


def kernel(feature1, feature2, output1, output2, target1, target2, label):
    B, D = output1.shape
    scalar = jax.ShapeDtypeStruct((1, 1), jnp.float32)

    full = lambda shape: pl.BlockSpec(shape, lambda p: (0, 0))
    pair, sse1, sse2 = pl.pallas_call(
        _body,
        grid=(1,),
        in_specs=[
            full((_H, D)),
            full((_H, D)),
            full((2, _H)),
            full((_H, D)),
            full((_H, D)),
            full((_H, D)),
            full((_H, D)),
        ],
        out_specs=[full((1, 1)), full((1, 1)), full((1, 1))],
        out_shape=[scalar, scalar, scalar],
        scratch_shapes=[pltpu.VMEM((1, _H), jnp.float32),
                        pltpu.VMEM((1, _H), jnp.float32)],
    )(feature1, feature2, label, output1, target1, output2, target2)

    n = 2 * _H
    n_pairs = jnp.float32(n * (n - 1) / 2)
    denom = jnp.float32(B * D)
    loss1 = sse1[0, 0] / denom
    loss2 = sse2[0, 0] / denom
    loss_mean = pair[0, 0] / n_pairs
    losses = loss_mean + (loss1 + loss2) / 2.0
    return (losses, loss1, loss2, loss_mean)
